# Initial kernel scaffold; baseline (speedup 1.0000x reference)
import jax
import jax.numpy as jnp
from jax.experimental import pallas as pl


def kernel(log_probs, target):
    raise NotImplementedError("write your pallas kernel here")


# speedup vs baseline: 2.5433x; 2.5433x over previous
import functools

import jax
import jax.numpy as jnp
from jax import lax
from jax.experimental import pallas as pl
from jax.experimental.pallas import tpu as pltpu
from jax.experimental.pallas import tpu_sc as plsc

_V = 32000
_N = 4096
_SMOOTHING = 0.1
_CONF = 1.0 - _SMOOTHING
_EPS = _SMOOTHING / (_V - 2)

import math as _math
_C = _SMOOTHING * _math.log(_EPS) + _CONF * _math.log(_CONF)

_R = 256
_CB = 6400
_NR = _N // _R
_NCB = _V // _CB

_NC = 2
_NS = 16
_NW = _NC * _NS
_RPW = _N // _NW
_LANES = 16
_CHUNKS = _RPW // _LANES


def _rowsum_body(x_ref, o_ref):
    j = pl.program_id(1)
    s = jnp.sum(x_ref[...], axis=1)

    @pl.when(j == 0)
    def _():
        o_ref[...] = s - x_ref[:, 0]

    @pl.when(j > 0)
    def _():
        o_ref[...] = o_ref[...] + s


def _rowsum_adj(log_probs):
    return pl.pallas_call(
        _rowsum_body,
        grid=(_NR, _NCB),
        in_specs=[pl.BlockSpec((_R, _CB), lambda i, j: (i, j))],
        out_specs=pl.BlockSpec((_R,), lambda i, j: (i,)),
        out_shape=jax.ShapeDtypeStruct((_N,), jnp.float32),
        compiler_params=pltpu.CompilerParams(
            dimension_semantics=("parallel", "arbitrary")
        ),
    )(log_probs)


@functools.cache
def _build_sc_partials():
    mesh = plsc.VectorSubcoreMesh(
        core_axis_name="c", subcore_axis_name="s", num_cores=_NC
    )

    @functools.partial(
        pl.kernel,
        mesh=mesh,
        out_type=jax.ShapeDtypeStruct((_NW, _LANES), jnp.float32),
        scratch_types=[
            pltpu.VMEM((_RPW,), jnp.int32),
            pltpu.VMEM((_RPW,), jnp.int32),
            pltpu.VMEM((_RPW,), jnp.float32),
            pltpu.VMEM((_RPW,), jnp.float32),
            pltpu.VMEM((_LANES,), jnp.float32),
            pltpu.SemaphoreType.DMA,
        ],
    )
    def _sc_partials(lp_flat_hbm, tgt_hbm, adj_hbm, out_hbm,
                     tgt_v, idx_v, adj_v, gat_v, acc_v, sem):
        wid = lax.axis_index("s") * _NC + lax.axis_index("c")
        base = wid * _RPW
        pltpu.sync_copy(tgt_hbm.at[pl.ds(base, _RPW)], tgt_v)
        pltpu.sync_copy(adj_hbm.at[pl.ds(base, _RPW)], adj_v)

        lane = lax.iota(jnp.int32, _LANES)
        for c in range(_CHUNKS):
            t16 = tgt_v[pl.ds(c * _LANES, _LANES)]
            row = (base + c * _LANES) + lane
            idx_v[pl.ds(c * _LANES, _LANES)] = row * _V + t16

        pltpu.async_copy(lp_flat_hbm.at[idx_v], gat_v, sem).wait()

        acc = jnp.zeros((_LANES,), jnp.float32)
        for c in range(_CHUNKS):
            t16 = tgt_v[pl.ds(c * _LANES, _LANES)]
            a16 = adj_v[pl.ds(c * _LANES, _LANES)]
            g16 = gat_v[pl.ds(c * _LANES, _LANES)]
            contrib = _C - _EPS * a16 - (_CONF - _EPS) * g16
            acc = acc + jnp.where(t16 != 0, contrib, jnp.float32(0.0))
        acc_v[...] = acc
        pltpu.sync_copy(acc_v, out_hbm.at[wid])

    return _sc_partials


def kernel(log_probs, target):
    adj = _rowsum_adj(log_probs)
    partials = _build_sc_partials()(
        log_probs.reshape(-1), target.astype(jnp.int32), adj
    )
    return jnp.sum(partials)


# speedup vs baseline: 2.5487x; 1.0021x over previous
import functools

import jax
import jax.numpy as jnp
from jax import lax
from jax.experimental import pallas as pl
from jax.experimental.pallas import tpu as pltpu
from jax.experimental.pallas import tpu_sc as plsc

_V = 32000
_N = 4096
_SMOOTHING = 0.1
_CONF = 1.0 - _SMOOTHING
_EPS = _SMOOTHING / (_V - 2)

import math as _math
_C = _SMOOTHING * _math.log(_EPS) + _CONF * _math.log(_CONF)

_R = 128
_CB = _V
_NR = _N // _R
_NCB = _V // _CB

_NC = 2
_NS = 16
_NW = _NC * _NS
_RPW = _N // _NW
_LANES = 16
_CHUNKS = _RPW // _LANES


def _rowsum_body(x_ref, o_ref):
    j = pl.program_id(1)
    s = jnp.sum(x_ref[...], axis=1)

    @pl.when(j == 0)
    def _():
        o_ref[...] = s - x_ref[:, 0]

    @pl.when(j > 0)
    def _():
        o_ref[...] = o_ref[...] + s


def _rowsum_adj(log_probs):
    return pl.pallas_call(
        _rowsum_body,
        grid=(_NR, _NCB),
        in_specs=[pl.BlockSpec((_R, _CB), lambda i, j: (i, j))],
        out_specs=pl.BlockSpec((_R,), lambda i, j: (i,)),
        out_shape=jax.ShapeDtypeStruct((_N,), jnp.float32),
        compiler_params=pltpu.CompilerParams(
            dimension_semantics=("parallel", "arbitrary")
        ),
    )(log_probs)


@functools.cache
def _build_sc_partials():
    mesh = plsc.VectorSubcoreMesh(
        core_axis_name="c", subcore_axis_name="s", num_cores=_NC
    )

    @functools.partial(
        pl.kernel,
        mesh=mesh,
        out_type=jax.ShapeDtypeStruct((_NW, _LANES), jnp.float32),
        scratch_types=[
            pltpu.VMEM((_RPW,), jnp.int32),
            pltpu.VMEM((_RPW,), jnp.int32),
            pltpu.VMEM((_RPW,), jnp.float32),
            pltpu.VMEM((_RPW,), jnp.float32),
            pltpu.VMEM((_LANES,), jnp.float32),
            pltpu.SemaphoreType.DMA,
        ],
    )
    def _sc_partials(lp_flat_hbm, tgt_hbm, adj_hbm, out_hbm,
                     tgt_v, idx_v, adj_v, gat_v, acc_v, sem):
        wid = lax.axis_index("s") * _NC + lax.axis_index("c")
        base = wid * _RPW
        pltpu.sync_copy(tgt_hbm.at[pl.ds(base, _RPW)], tgt_v)
        pltpu.sync_copy(adj_hbm.at[pl.ds(base, _RPW)], adj_v)

        lane = lax.iota(jnp.int32, _LANES)
        for c in range(_CHUNKS):
            t16 = tgt_v[pl.ds(c * _LANES, _LANES)]
            row = (base + c * _LANES) + lane
            idx_v[pl.ds(c * _LANES, _LANES)] = row * _V + t16

        pltpu.async_copy(lp_flat_hbm.at[idx_v], gat_v, sem).wait()

        acc = jnp.zeros((_LANES,), jnp.float32)
        for c in range(_CHUNKS):
            t16 = tgt_v[pl.ds(c * _LANES, _LANES)]
            a16 = adj_v[pl.ds(c * _LANES, _LANES)]
            g16 = gat_v[pl.ds(c * _LANES, _LANES)]
            contrib = _C - _EPS * a16 - (_CONF - _EPS) * g16
            acc = acc + jnp.where(t16 != 0, contrib, jnp.float32(0.0))
        acc_v[...] = acc
        pltpu.sync_copy(acc_v, out_hbm.at[wid])

    return _sc_partials


def kernel(log_probs, target):
    adj = _rowsum_adj(log_probs)
    partials = _build_sc_partials()(
        log_probs.reshape(-1), target.astype(jnp.int32), adj
    )
    return jnp.sum(partials)
